# baseline (speedup 1.0000x reference)
import functools
import jax
import jax.numpy as jnp
from jax.experimental import pallas as pl
from jax.experimental.pallas import tpu as pltpu

N_NODES = 50000
N_EDGES = 800000
_LOG2 = 0.6931471805599453

_INTERP = False


def _sp2(x):
    return jax.nn.softplus(x) - _LOG2


def _mlp_rows(ps, x, activate_last=True):
    n = len(ps)
    for i, (W, b) in enumerate(ps):
        x = x @ W + b
        if i < n - 1 or activate_last:
            x = _sp2(x)
    return x


def _mlp2(x, p, T):
    (W1, b1), (W2, b2) = p
    N, Din = x.shape
    H = W1.shape[1]
    Dout = W2.shape[1]

    def body(x_ref, w1_ref, b1_ref, w2_ref, b2_ref, o_ref):
        h = _sp2(jnp.dot(x_ref[...], w1_ref[...],
                         preferred_element_type=jnp.float32) + b1_ref[...])
        o_ref[...] = _sp2(jnp.dot(h, w2_ref[...],
                                  preferred_element_type=jnp.float32) + b2_ref[...])

    return pl.pallas_call(
        body,
        grid=(N // T,),
        in_specs=[
            pl.BlockSpec((T, Din), lambda i: (i, 0)),
            pl.BlockSpec((Din, H), lambda i: (0, 0)),
            pl.BlockSpec((1, H), lambda i: (0, 0)),
            pl.BlockSpec((H, Dout), lambda i: (0, 0)),
            pl.BlockSpec((1, Dout), lambda i: (0, 0)),
        ],
        out_specs=pl.BlockSpec((T, Dout), lambda i: (i, 0)),
        out_shape=jax.ShapeDtypeStruct((N, Dout), jnp.float32),
        interpret=_INTERP,
    )(x, W1, b1.reshape(1, -1), W2, b2.reshape(1, -1))


def _node_encode(node_feat, emb, p, T=2000):
    (W1, b1), (W2, b2) = p
    n_elem, demb = emb.shape
    H = W1.shape[1]
    Dout = W2.shape[1]
    nf = node_feat.astype(jnp.int32).reshape(N_NODES, 1)

    def body(f_ref, emb_ref, w1_ref, b1_ref, w2_ref, b2_ref, o_ref):
        f = f_ref[...]
        ids = jax.lax.broadcasted_iota(jnp.int32, (T, n_elem), 1)
        oh = (f == ids).astype(jnp.float32)
        x = jnp.dot(oh, emb_ref[...], preferred_element_type=jnp.float32)
        h = _sp2(jnp.dot(x, w1_ref[...],
                         preferred_element_type=jnp.float32) + b1_ref[...])
        o_ref[...] = _sp2(jnp.dot(h, w2_ref[...],
                                  preferred_element_type=jnp.float32) + b2_ref[...])

    return pl.pallas_call(
        body,
        grid=(N_NODES // T,),
        in_specs=[
            pl.BlockSpec((T, 1), lambda i: (i, 0)),
            pl.BlockSpec((n_elem, demb), lambda i: (0, 0)),
            pl.BlockSpec((demb, H), lambda i: (0, 0)),
            pl.BlockSpec((1, H), lambda i: (0, 0)),
            pl.BlockSpec((H, Dout), lambda i: (0, 0)),
            pl.BlockSpec((1, Dout), lambda i: (0, 0)),
        ],
        out_specs=pl.BlockSpec((T, Dout), lambda i: (i, 0)),
        out_shape=jax.ShapeDtypeStruct((N_NODES, Dout), jnp.float32),
        interpret=_INTERP,
    )(nf, emb, W1, b1.reshape(1, -1), W2, b2.reshape(1, -1))


def _conv_edge(e0, vi, vj, c1, pre, cw, T=4000):
    (W1, _b1), (W2, b2), (W3, b3) = cw
    W1cut = W1[0:96]
    N = e0.shape[0]

    def body(*refs):
        if pre is not None:
            (e0_ref, vi_ref, vj_ref, c1_ref, w1_ref, w2_ref, b2_ref, w3_ref,
             b3_ref, p1w, p1b, p2w, p2b, ec_ref, er_ref, ms_ref) = refs
        else:
            (e0_ref, vi_ref, vj_ref, c1_ref, w1_ref, w2_ref, b2_ref, w3_ref,
             b3_ref, ec_ref, er_ref, ms_ref) = refs
        e0b = e0_ref[...]
        ep = e0b
        if pre is not None:
            ep = _sp2(jnp.dot(ep, p1w[...],
                              preferred_element_type=jnp.float32) + p1b[...])
            ep = _sp2(jnp.dot(ep, p2w[...],
                              preferred_element_type=jnp.float32) + p2b[...])
        x = jnp.concatenate([vi_ref[...], vj_ref[...], ep], axis=1)
        h = _sp2(jnp.dot(x, w1_ref[...],
                         preferred_element_type=jnp.float32) + c1_ref[...])
        h = _sp2(jnp.dot(h, w2_ref[...],
                         preferred_element_type=jnp.float32) + b2_ref[...])
        ec = _sp2(jnp.dot(h, w3_ref[...],
                          preferred_element_type=jnp.float32) + b3_ref[...])
        ec_ref[...] = ec
        er_ref[...] = ec + e0b
        i = pl.program_id(0)

        @pl.when(i == 0)
        def _():
            ms_ref[...] = jnp.zeros_like(ms_ref)

        ms_ref[...] += jnp.broadcast_to(
            jnp.sum(ec, axis=0, keepdims=True), ms_ref.shape)

    args = [e0, vi, vj, c1, W1cut, W2, b2.reshape(1, -1), W3, b3.reshape(1, -1)]
    in_specs = [
        pl.BlockSpec((T, 32), lambda i: (i, 0)),
        pl.BlockSpec((T, 32), lambda i: (i, 0)),
        pl.BlockSpec((T, 32), lambda i: (i, 0)),
        pl.BlockSpec((1, 64), lambda i: (0, 0)),
        pl.BlockSpec((96, 64), lambda i: (0, 0)),
        pl.BlockSpec((64, 64), lambda i: (0, 0)),
        pl.BlockSpec((1, 64), lambda i: (0, 0)),
        pl.BlockSpec((64, 32), lambda i: (0, 0)),
        pl.BlockSpec((1, 32), lambda i: (0, 0)),
    ]
    if pre is not None:
        (P1w, P1b), (P2w, P2b) = pre
        args += [P1w, P1b.reshape(1, -1), P2w, P2b.reshape(1, -1)]
        in_specs += [
            pl.BlockSpec((32, 64), lambda i: (0, 0)),
            pl.BlockSpec((1, 64), lambda i: (0, 0)),
            pl.BlockSpec((64, 32), lambda i: (0, 0)),
            pl.BlockSpec((1, 32), lambda i: (0, 0)),
        ]
    ec, er, ms = pl.pallas_call(
        body,
        grid=(N // T,),
        in_specs=in_specs,
        out_specs=[
            pl.BlockSpec((T, 32), lambda i: (i, 0)),
            pl.BlockSpec((T, 32), lambda i: (i, 0)),
            pl.BlockSpec((8, 32), lambda i: (0, 0)),
        ],
        out_shape=[
            jax.ShapeDtypeStruct((N, 32), jnp.float32),
            jax.ShapeDtypeStruct((N, 32), jnp.float32),
            jax.ShapeDtypeStruct((8, 32), jnp.float32),
        ],
        interpret=_INTERP,
    )(*args)
    return ec, er, ms[0:1]


def _conv_node(v0, vp, esA, esB, cntA, cntB, cn1, cw, T=2000):
    (W1, _b1), (W2, b2), (W3, b3) = cw
    W1cut = W1[0:64]

    def body(v0_ref, vp_ref, ea_ref, eb_ref, ca_ref, cb_ref, c1_ref,
             w1_ref, w2_ref, b2_ref, w3_ref, b3_ref, vr_ref, ms_ref):
        cnt = ca_ref[:, 0:1] + cb_ref[:, 0:1]
        ve = (ea_ref[...] + eb_ref[...]) / jnp.maximum(cnt, 1.0)
        x = jnp.concatenate([vp_ref[...], ve], axis=1)
        h = _sp2(jnp.dot(x, w1_ref[...],
                         preferred_element_type=jnp.float32) + c1_ref[...])
        h = _sp2(jnp.dot(h, w2_ref[...],
                         preferred_element_type=jnp.float32) + b2_ref[...])
        vc = _sp2(jnp.dot(h, w3_ref[...],
                          preferred_element_type=jnp.float32) + b3_ref[...])
        vr_ref[...] = vc + v0_ref[...]
        i = pl.program_id(0)

        @pl.when(i == 0)
        def _():
            ms_ref[...] = jnp.zeros_like(ms_ref)

        ms_ref[...] += jnp.broadcast_to(
            jnp.sum(vc, axis=0, keepdims=True), ms_ref.shape)

    vr, ms = pl.pallas_call(
        body,
        grid=(N_NODES // T,),
        in_specs=[
            pl.BlockSpec((T, 32), lambda i: (i, 0)),
            pl.BlockSpec((T, 32), lambda i: (i, 0)),
            pl.BlockSpec((T, 32), lambda i: (i, 0)),
            pl.BlockSpec((T, 32), lambda i: (i, 0)),
            pl.BlockSpec((T, 16), lambda i: (i, 0)),
            pl.BlockSpec((T, 16), lambda i: (i, 0)),
            pl.BlockSpec((1, 64), lambda i: (0, 0)),
            pl.BlockSpec((64, 64), lambda i: (0, 0)),
            pl.BlockSpec((64, 64), lambda i: (0, 0)),
            pl.BlockSpec((1, 64), lambda i: (0, 0)),
            pl.BlockSpec((64, 32), lambda i: (0, 0)),
            pl.BlockSpec((1, 32), lambda i: (0, 0)),
        ],
        out_specs=[
            pl.BlockSpec((T, 32), lambda i: (i, 0)),
            pl.BlockSpec((8, 32), lambda i: (0, 0)),
        ],
        out_shape=[
            jax.ShapeDtypeStruct((N_NODES, 32), jnp.float32),
            jax.ShapeDtypeStruct((8, 32), jnp.float32),
        ],
        interpret=_INTERP,
    )(v0, vp, esA, esB, cntA, cntB, cn1,
      W1cut, W2, b2.reshape(1, -1), W3, b3.reshape(1, -1))
    return vr, ms[0:1]


def _s2s_max(feat, q, T):
    N = feat.shape[0]

    def body(f_ref, q_ref, m_ref):
        en = jnp.sum(f_ref[...] * q_ref[...], axis=1, keepdims=True)
        mt = jnp.max(en, axis=0, keepdims=True)
        i = pl.program_id(0)

        @pl.when(i == 0)
        def _():
            m_ref[...] = jnp.full_like(m_ref, -jnp.inf)

        m_ref[...] = jnp.maximum(m_ref[...],
                                 jnp.broadcast_to(mt, m_ref.shape))

    return pl.pallas_call(
        body,
        grid=(N // T,),
        in_specs=[
            pl.BlockSpec((T, 32), lambda i: (i, 0)),
            pl.BlockSpec((1, 32), lambda i: (0, 0)),
        ],
        out_specs=pl.BlockSpec((8, 128), lambda i: (0, 0)),
        out_shape=jax.ShapeDtypeStruct((8, 128), jnp.float32),
        interpret=_INTERP,
    )(feat, q)


def _s2s_weighted(feat, q, m, T):
    N = feat.shape[0]

    def body(f_ref, q_ref, m_ref, r_ref, s_ref):
        f = f_ref[...]
        en = jnp.sum(f * q_ref[...], axis=1, keepdims=True)
        w = jnp.exp(en - m_ref[0:1, 0:1])
        r = jnp.sum(w * f, axis=0, keepdims=True)
        s = jnp.sum(w, axis=0, keepdims=True)
        i = pl.program_id(0)

        @pl.when(i == 0)
        def _():
            r_ref[...] = jnp.zeros_like(r_ref)
            s_ref[...] = jnp.zeros_like(s_ref)

        r_ref[...] += jnp.broadcast_to(r, r_ref.shape)
        s_ref[...] += jnp.broadcast_to(s, s_ref.shape)

    return pl.pallas_call(
        body,
        grid=(N // T,),
        in_specs=[
            pl.BlockSpec((T, 32), lambda i: (i, 0)),
            pl.BlockSpec((1, 32), lambda i: (0, 0)),
            pl.BlockSpec((8, 128), lambda i: (0, 0)),
        ],
        out_specs=[
            pl.BlockSpec((8, 32), lambda i: (0, 0)),
            pl.BlockSpec((8, 128), lambda i: (0, 0)),
        ],
        out_shape=[
            jax.ShapeDtypeStruct((8, 32), jnp.float32),
            jax.ShapeDtypeStruct((8, 128), jnp.float32),
        ],
        interpret=_INTERP,
    )(feat, q, m)


def _lstm(p, x, h, c):
    W_ih, W_hh, b_ih, b_hh = p
    gates = x @ W_ih.T + b_ih + h @ W_hh.T + b_hh
    i, f, g, o = jnp.split(gates, 4, axis=-1)
    c = jax.nn.sigmoid(f) * c + jax.nn.sigmoid(i) * jnp.tanh(g)
    h = jax.nn.sigmoid(o) * jnp.tanh(c)
    return h, c


def _set2set(p, feat, T):
    D = feat.shape[1]
    q_star = jnp.zeros((1, 2 * D), jnp.float32)
    h = jnp.zeros((1, D), jnp.float32)
    c = jnp.zeros((1, D), jnp.float32)
    for _ in range(2):
        h, c = _lstm(p, q_star, h, c)
        m = _s2s_max(feat, h, T)
        r, s = _s2s_weighted(feat, h, m, T)
        readout = r[0:1] / s[0, 0]
        q_star = jnp.concatenate([h, readout], axis=-1)
    return q_star


def _gather_rows(table, idx):
    return table[idx]


def _scatter_add_32(rows, idx):
    z = jax.ops.segment_sum(rows, idx, num_segments=N_NODES)
    return z, jnp.zeros_like(z)


def _seg_counts(idx):
    c = jax.ops.segment_sum(jnp.ones((N_EDGES, 1), jnp.float32), idx,
                            num_segments=N_NODES)
    c16 = jnp.broadcast_to(c, (N_NODES, 16))
    return c16, jnp.zeros_like(c16)


def kernel(edge_feat, node_feat, edge_index, graph_attr, params):
    src = edge_index[0].astype(jnp.int32)
    dst = edge_index[1].astype(jnp.int32)

    e = _mlp2(edge_feat, params['edge_encoder'], T=4000)
    v = _node_encode(node_feat, params['node_embedding'],
                     params['node_encoder'])
    u = _mlp_rows(params['attr_encoder'], graph_attr)

    cntA, cntB = _seg_counts(dst)

    for blk in params['blocks']:
        e0, v0, u0 = e, v, u
        if blk['pre_edge'] is not None:
            vp = _mlp2(v, blk['pre_node'], T=2000)
            up = _mlp_rows(blk['pre_attr'], u)
            pre_e = blk['pre_edge']
        else:
            vp, up, pre_e = v, u, None
        vi = _gather_rows(vp, src)
        vj = _gather_rows(vp, dst)
        (W1, b1) = blk['conv_edge'][0]
        c1 = up @ W1[96:128] + b1
        ec, e, me_sum = _conv_edge(e0, vi, vj, c1, pre_e, blk['conv_edge'])
        esA, esB = _scatter_add_32(ec, dst)
        (NW1, nb1) = blk['conv_node'][0]
        cn1 = up @ NW1[64:96] + nb1
        v, mv_sum = _conv_node(v0, vp, esA, esB, cntA, cntB, cn1,
                               blk['conv_node'])
        mean_v = mv_sum / N_NODES
        mean_e = me_sum / N_EDGES
        uc = _mlp_rows(blk['conv_attr'],
                       jnp.concatenate([up, mean_v, mean_e], axis=-1))
        u = uc + u0

    node_vec = _set2set(params['node_s2s'], v, T=2000)
    edge_vec = _set2set(params['edge_s2s'], e, T=4000)
    vec = jnp.concatenate([node_vec, edge_vec, u], axis=-1)
    out = _mlp_rows(params['output_proj'], vec, activate_last=False)
    return jnp.squeeze(out)


# speedup vs baseline: 1.9989x; 1.9989x over previous
import functools
import jax
import jax.numpy as jnp
from jax import lax
from jax.experimental import pallas as pl
from jax.experimental.pallas import tpu as pltpu
from jax.experimental.pallas import tpu_sc as plsc

N_NODES = 50000
N_EDGES = 800000
_LOG2 = 0.6931471805599453

_INTERP = False


def _sp2(x):
    return jax.nn.softplus(x) - _LOG2


def _mlp_rows(ps, x, activate_last=True):
    n = len(ps)
    for i, (W, b) in enumerate(ps):
        x = x @ W + b
        if i < n - 1 or activate_last:
            x = _sp2(x)
    return x


def _mlp2(x, p, T):
    (W1, b1), (W2, b2) = p
    N, Din = x.shape
    H = W1.shape[1]
    Dout = W2.shape[1]

    def body(x_ref, w1_ref, b1_ref, w2_ref, b2_ref, o_ref):
        h = _sp2(jnp.dot(x_ref[...], w1_ref[...],
                         preferred_element_type=jnp.float32) + b1_ref[...])
        o_ref[...] = _sp2(jnp.dot(h, w2_ref[...],
                                  preferred_element_type=jnp.float32) + b2_ref[...])

    return pl.pallas_call(
        body,
        grid=(N // T,),
        in_specs=[
            pl.BlockSpec((T, Din), lambda i: (i, 0)),
            pl.BlockSpec((Din, H), lambda i: (0, 0)),
            pl.BlockSpec((1, H), lambda i: (0, 0)),
            pl.BlockSpec((H, Dout), lambda i: (0, 0)),
            pl.BlockSpec((1, Dout), lambda i: (0, 0)),
        ],
        out_specs=pl.BlockSpec((T, Dout), lambda i: (i, 0)),
        out_shape=jax.ShapeDtypeStruct((N, Dout), jnp.float32),
        interpret=_INTERP,
    )(x, W1, b1.reshape(1, -1), W2, b2.reshape(1, -1))


def _node_encode(node_feat, emb, p, T=2000):
    (W1, b1), (W2, b2) = p
    n_elem, demb = emb.shape
    H = W1.shape[1]
    Dout = W2.shape[1]
    nf = node_feat.astype(jnp.int32).reshape(N_NODES, 1)

    def body(f_ref, emb_ref, w1_ref, b1_ref, w2_ref, b2_ref, o_ref):
        f = f_ref[...]
        ids = jax.lax.broadcasted_iota(jnp.int32, (T, n_elem), 1)
        oh = (f == ids).astype(jnp.float32)
        x = jnp.dot(oh, emb_ref[...], preferred_element_type=jnp.float32)
        h = _sp2(jnp.dot(x, w1_ref[...],
                         preferred_element_type=jnp.float32) + b1_ref[...])
        o_ref[...] = _sp2(jnp.dot(h, w2_ref[...],
                                  preferred_element_type=jnp.float32) + b2_ref[...])

    return pl.pallas_call(
        body,
        grid=(N_NODES // T,),
        in_specs=[
            pl.BlockSpec((T, 1), lambda i: (i, 0)),
            pl.BlockSpec((n_elem, demb), lambda i: (0, 0)),
            pl.BlockSpec((demb, H), lambda i: (0, 0)),
            pl.BlockSpec((1, H), lambda i: (0, 0)),
            pl.BlockSpec((H, Dout), lambda i: (0, 0)),
            pl.BlockSpec((1, Dout), lambda i: (0, 0)),
        ],
        out_specs=pl.BlockSpec((T, Dout), lambda i: (i, 0)),
        out_shape=jax.ShapeDtypeStruct((N_NODES, Dout), jnp.float32),
        interpret=_INTERP,
    )(nf, emb, W1, b1.reshape(1, -1), W2, b2.reshape(1, -1))


def _conv_edge(e0, vi, vj, c1, pre, cw, T=4000):
    (W1, _b1), (W2, b2), (W3, b3) = cw
    W1cut = W1[0:96]
    N = e0.shape[0]

    def body(*refs):
        if pre is not None:
            (e0_ref, vi_ref, vj_ref, c1_ref, w1_ref, w2_ref, b2_ref, w3_ref,
             b3_ref, p1w, p1b, p2w, p2b, ec_ref, er_ref, ms_ref) = refs
        else:
            (e0_ref, vi_ref, vj_ref, c1_ref, w1_ref, w2_ref, b2_ref, w3_ref,
             b3_ref, ec_ref, er_ref, ms_ref) = refs
        e0b = e0_ref[...]
        ep = e0b
        if pre is not None:
            ep = _sp2(jnp.dot(ep, p1w[...],
                              preferred_element_type=jnp.float32) + p1b[...])
            ep = _sp2(jnp.dot(ep, p2w[...],
                              preferred_element_type=jnp.float32) + p2b[...])
        x = jnp.concatenate([vi_ref[...], vj_ref[...], ep], axis=1)
        h = _sp2(jnp.dot(x, w1_ref[...],
                         preferred_element_type=jnp.float32) + c1_ref[...])
        h = _sp2(jnp.dot(h, w2_ref[...],
                         preferred_element_type=jnp.float32) + b2_ref[...])
        ec = _sp2(jnp.dot(h, w3_ref[...],
                          preferred_element_type=jnp.float32) + b3_ref[...])
        ec_ref[...] = ec
        er_ref[...] = ec + e0b
        i = pl.program_id(0)

        @pl.when(i == 0)
        def _():
            ms_ref[...] = jnp.zeros_like(ms_ref)

        ms_ref[...] += jnp.broadcast_to(
            jnp.sum(ec, axis=0, keepdims=True), ms_ref.shape)

    args = [e0, vi, vj, c1, W1cut, W2, b2.reshape(1, -1), W3, b3.reshape(1, -1)]
    in_specs = [
        pl.BlockSpec((T, 32), lambda i: (i, 0)),
        pl.BlockSpec((T, 32), lambda i: (i, 0)),
        pl.BlockSpec((T, 32), lambda i: (i, 0)),
        pl.BlockSpec((1, 64), lambda i: (0, 0)),
        pl.BlockSpec((96, 64), lambda i: (0, 0)),
        pl.BlockSpec((64, 64), lambda i: (0, 0)),
        pl.BlockSpec((1, 64), lambda i: (0, 0)),
        pl.BlockSpec((64, 32), lambda i: (0, 0)),
        pl.BlockSpec((1, 32), lambda i: (0, 0)),
    ]
    if pre is not None:
        (P1w, P1b), (P2w, P2b) = pre
        args += [P1w, P1b.reshape(1, -1), P2w, P2b.reshape(1, -1)]
        in_specs += [
            pl.BlockSpec((32, 64), lambda i: (0, 0)),
            pl.BlockSpec((1, 64), lambda i: (0, 0)),
            pl.BlockSpec((64, 32), lambda i: (0, 0)),
            pl.BlockSpec((1, 32), lambda i: (0, 0)),
        ]
    ec, er, ms = pl.pallas_call(
        body,
        grid=(N // T,),
        in_specs=in_specs,
        out_specs=[
            pl.BlockSpec((T, 32), lambda i: (i, 0)),
            pl.BlockSpec((T, 32), lambda i: (i, 0)),
            pl.BlockSpec((8, 32), lambda i: (0, 0)),
        ],
        out_shape=[
            jax.ShapeDtypeStruct((N, 32), jnp.float32),
            jax.ShapeDtypeStruct((N, 32), jnp.float32),
            jax.ShapeDtypeStruct((8, 32), jnp.float32),
        ],
        interpret=_INTERP,
    )(*args)
    return ec, er, ms[0:1]


def _conv_node(v0, vp, esA, esB, cntA, cntB, cn1, cw, T=2000):
    (W1, _b1), (W2, b2), (W3, b3) = cw
    W1cut = W1[0:64]

    def body(v0_ref, vp_ref, ea_ref, eb_ref, ca_ref, cb_ref, c1_ref,
             w1_ref, w2_ref, b2_ref, w3_ref, b3_ref, vr_ref, ms_ref):
        cnt = ca_ref[:, 0:1] + cb_ref[:, 0:1]
        ve = (ea_ref[...] + eb_ref[...]) / jnp.maximum(cnt, 1.0)
        x = jnp.concatenate([vp_ref[...], ve], axis=1)
        h = _sp2(jnp.dot(x, w1_ref[...],
                         preferred_element_type=jnp.float32) + c1_ref[...])
        h = _sp2(jnp.dot(h, w2_ref[...],
                         preferred_element_type=jnp.float32) + b2_ref[...])
        vc = _sp2(jnp.dot(h, w3_ref[...],
                          preferred_element_type=jnp.float32) + b3_ref[...])
        vr_ref[...] = vc + v0_ref[...]
        i = pl.program_id(0)

        @pl.when(i == 0)
        def _():
            ms_ref[...] = jnp.zeros_like(ms_ref)

        ms_ref[...] += jnp.broadcast_to(
            jnp.sum(vc, axis=0, keepdims=True), ms_ref.shape)

    vr, ms = pl.pallas_call(
        body,
        grid=(N_NODES // T,),
        in_specs=[
            pl.BlockSpec((T, 32), lambda i: (i, 0)),
            pl.BlockSpec((T, 32), lambda i: (i, 0)),
            pl.BlockSpec((T, 32), lambda i: (i, 0)),
            pl.BlockSpec((T, 32), lambda i: (i, 0)),
            pl.BlockSpec((T, 16), lambda i: (i, 0)),
            pl.BlockSpec((T, 16), lambda i: (i, 0)),
            pl.BlockSpec((1, 64), lambda i: (0, 0)),
            pl.BlockSpec((64, 64), lambda i: (0, 0)),
            pl.BlockSpec((64, 64), lambda i: (0, 0)),
            pl.BlockSpec((1, 64), lambda i: (0, 0)),
            pl.BlockSpec((64, 32), lambda i: (0, 0)),
            pl.BlockSpec((1, 32), lambda i: (0, 0)),
        ],
        out_specs=[
            pl.BlockSpec((T, 32), lambda i: (i, 0)),
            pl.BlockSpec((8, 32), lambda i: (0, 0)),
        ],
        out_shape=[
            jax.ShapeDtypeStruct((N_NODES, 32), jnp.float32),
            jax.ShapeDtypeStruct((8, 32), jnp.float32),
        ],
        interpret=_INTERP,
    )(v0, vp, esA, esB, cntA, cntB, cn1,
      W1cut, W2, b2.reshape(1, -1), W3, b3.reshape(1, -1))
    return vr, ms[0:1]


def _s2s_max(feat, q, T):
    N = feat.shape[0]

    def body(f_ref, q_ref, m_ref):
        en = jnp.sum(f_ref[...] * q_ref[...], axis=1, keepdims=True)
        mt = jnp.max(en, axis=0, keepdims=True)
        i = pl.program_id(0)

        @pl.when(i == 0)
        def _():
            m_ref[...] = jnp.full_like(m_ref, -jnp.inf)

        m_ref[...] = jnp.maximum(m_ref[...],
                                 jnp.broadcast_to(mt, m_ref.shape))

    return pl.pallas_call(
        body,
        grid=(N // T,),
        in_specs=[
            pl.BlockSpec((T, 32), lambda i: (i, 0)),
            pl.BlockSpec((1, 32), lambda i: (0, 0)),
        ],
        out_specs=pl.BlockSpec((8, 128), lambda i: (0, 0)),
        out_shape=jax.ShapeDtypeStruct((8, 128), jnp.float32),
        interpret=_INTERP,
    )(feat, q)


def _s2s_weighted(feat, q, m, T):
    N = feat.shape[0]

    def body(f_ref, q_ref, m_ref, r_ref, s_ref):
        f = f_ref[...]
        en = jnp.sum(f * q_ref[...], axis=1, keepdims=True)
        w = jnp.exp(en - m_ref[0:1, 0:1])
        r = jnp.sum(w * f, axis=0, keepdims=True)
        s = jnp.sum(w, axis=0, keepdims=True)
        i = pl.program_id(0)

        @pl.when(i == 0)
        def _():
            r_ref[...] = jnp.zeros_like(r_ref)
            s_ref[...] = jnp.zeros_like(s_ref)

        r_ref[...] += jnp.broadcast_to(r, r_ref.shape)
        s_ref[...] += jnp.broadcast_to(s, s_ref.shape)

    return pl.pallas_call(
        body,
        grid=(N // T,),
        in_specs=[
            pl.BlockSpec((T, 32), lambda i: (i, 0)),
            pl.BlockSpec((1, 32), lambda i: (0, 0)),
            pl.BlockSpec((8, 128), lambda i: (0, 0)),
        ],
        out_specs=[
            pl.BlockSpec((8, 32), lambda i: (0, 0)),
            pl.BlockSpec((8, 128), lambda i: (0, 0)),
        ],
        out_shape=[
            jax.ShapeDtypeStruct((8, 32), jnp.float32),
            jax.ShapeDtypeStruct((8, 128), jnp.float32),
        ],
        interpret=_INTERP,
    )(feat, q, m)


def _lstm(p, x, h, c):
    W_ih, W_hh, b_ih, b_hh = p
    gates = x @ W_ih.T + b_ih + h @ W_hh.T + b_hh
    i, f, g, o = jnp.split(gates, 4, axis=-1)
    c = jax.nn.sigmoid(f) * c + jax.nn.sigmoid(i) * jnp.tanh(g)
    h = jax.nn.sigmoid(o) * jnp.tanh(c)
    return h, c


def _set2set(p, feat, T):
    D = feat.shape[1]
    q_star = jnp.zeros((1, 2 * D), jnp.float32)
    h = jnp.zeros((1, D), jnp.float32)
    c = jnp.zeros((1, D), jnp.float32)
    for _ in range(2):
        h, c = _lstm(p, q_star, h, c)
        m = _s2s_max(feat, h, T)
        r, s = _s2s_weighted(feat, h, m, T)
        readout = r[0:1] / s[0, 0]
        q_star = jnp.concatenate([h, readout], axis=-1)
    return q_star


_NW = 32
_C = 128
_NCH = N_EDGES // _C
_JMAX = -(-_NCH // _NW)
_NB = N_NODES // 1000


def _wid():
    return lax.axis_index("s") * 2 + lax.axis_index("c")


def _edge_gather(vp, src, dst):
    mesh = plsc.VectorSubcoreMesh(core_axis_name="c", subcore_axis_name="s")

    @functools.partial(
        pl.kernel, mesh=mesh,
        out_type=[jax.ShapeDtypeStruct((N_EDGES, 32), jnp.float32),
                  jax.ShapeDtypeStruct((N_EDGES, 32), jnp.float32)],
        scratch_types=[
            pltpu.VMEM((_C,), jnp.int32),
            pltpu.VMEM((_C,), jnp.int32),
            pltpu.VMEM((_C, 32), jnp.float32),
            pltpu.VMEM((_C, 32), jnp.float32),
            pltpu.SemaphoreType.DMA,
            pltpu.SemaphoreType.DMA,
        ],
        compiler_params=pltpu.CompilerParams(use_tc_tiling_on_sc=False),
    )
    def k(vp_hbm, src_hbm, dst_hbm, vi_hbm, vj_hbm,
          si_v, di_v, ri_v, rj_v, sem1, sem2):
        w = _wid()

        def body(j, _):
            c = w + _NW * j

            @pl.when(c < _NCH)
            def _():
                off = c * _C
                pltpu.sync_copy(src_hbm.at[pl.ds(off, _C)], si_v)
                pltpu.sync_copy(dst_hbm.at[pl.ds(off, _C)], di_v)
                cp1 = pltpu.async_copy(vp_hbm.at[si_v], ri_v, sem1)
                cp2 = pltpu.async_copy(vp_hbm.at[di_v], rj_v, sem2)
                cp1.wait()
                pltpu.sync_copy(ri_v, vi_hbm.at[pl.ds(off, _C)])
                cp2.wait()
                pltpu.sync_copy(rj_v, vj_hbm.at[pl.ds(off, _C)])
            return 0

        lax.fori_loop(0, _JMAX, body, 0)

    return k(vp, src, dst)


def _sc_scatter_body(rows_hbm_or_none, idx_hbm, out_hbm, z_hbm, ones_hbm,
                     idx_v, rows_v, shared):
    w = _wid()
    sid = lax.axis_index("s")
    cid = lax.axis_index("c")

    if rows_hbm_or_none is None:
        pltpu.sync_copy(ones_hbm, rows_v)

    for ci in range(4):
        b = sid + 16 * ci

        @pl.when(b < _NB)
        def _():
            pltpu.sync_copy(z_hbm, shared.at[pl.ds(b * 1000, 1000)])
    plsc.subcore_barrier()

    def body(j, _):
        c = w + _NW * j

        @pl.when(c < _NCH)
        def _():
            off = c * _C
            pltpu.sync_copy(idx_hbm.at[pl.ds(off, _C)], idx_v)
            if rows_hbm_or_none is not None:
                pltpu.sync_copy(rows_hbm_or_none.at[pl.ds(off, _C)], rows_v)
            pltpu.sync_copy(rows_v, shared.at[idx_v], add=True)
        return 0

    lax.fori_loop(0, _JMAX, body, 0)
    plsc.subcore_barrier()

    for ci in range(4):
        b = sid + 16 * ci

        @pl.when(b < _NB)
        def _():
            pltpu.sync_copy(shared.at[pl.ds(b * 1000, 1000)],
                            out_hbm.at[cid, pl.ds(b * 1000, 1000)])


def _scatter_add_32(rows, idx):
    mesh = plsc.VectorSubcoreMesh(core_axis_name="c", subcore_axis_name="s")
    z = jnp.zeros((1000, 32), jnp.float32)
    ones = jnp.ones((_C, 32), jnp.float32)

    @functools.partial(
        pl.kernel, mesh=mesh,
        out_type=jax.ShapeDtypeStruct((2, N_NODES, 32), jnp.float32),
        scratch_types=[
            pltpu.VMEM((_C,), jnp.int32),
            pltpu.VMEM((_C, 32), jnp.float32),
            pltpu.VMEM_SHARED((N_NODES, 32), jnp.float32),
        ],
        compiler_params=pltpu.CompilerParams(use_tc_tiling_on_sc=False),
    )
    def k(rows_hbm, idx_hbm, z_hbm, ones_hbm, out_hbm, idx_v, rows_v, shared):
        _sc_scatter_body(rows_hbm, idx_hbm, out_hbm, z_hbm, ones_hbm,
                         idx_v, rows_v, shared)

    out = k(rows, idx, z, ones)
    return out[0], out[1]


def _seg_counts(idx):
    mesh = plsc.VectorSubcoreMesh(core_axis_name="c", subcore_axis_name="s")
    z = jnp.zeros((1000, 16), jnp.float32)
    ones = jnp.ones((_C, 16), jnp.float32)

    @functools.partial(
        pl.kernel, mesh=mesh,
        out_type=jax.ShapeDtypeStruct((2, N_NODES, 16), jnp.float32),
        scratch_types=[
            pltpu.VMEM((_C,), jnp.int32),
            pltpu.VMEM((_C, 16), jnp.float32),
            pltpu.VMEM_SHARED((N_NODES, 16), jnp.float32),
        ],
        compiler_params=pltpu.CompilerParams(use_tc_tiling_on_sc=False),
    )
    def k(idx_hbm, z_hbm, ones_hbm, out_hbm, idx_v, rows_v, shared):
        _sc_scatter_body(None, idx_hbm, out_hbm, z_hbm, ones_hbm,
                         idx_v, rows_v, shared)

    out = k(idx, z, ones)
    return out[0], out[1]


def kernel(edge_feat, node_feat, edge_index, graph_attr, params):
    src = edge_index[0].astype(jnp.int32)
    dst = edge_index[1].astype(jnp.int32)

    e = _mlp2(edge_feat, params['edge_encoder'], T=4000)
    v = _node_encode(node_feat, params['node_embedding'],
                     params['node_encoder'])
    u = _mlp_rows(params['attr_encoder'], graph_attr)

    cntA, cntB = _seg_counts(dst)

    for blk in params['blocks']:
        e0, v0, u0 = e, v, u
        if blk['pre_edge'] is not None:
            vp = _mlp2(v, blk['pre_node'], T=2000)
            up = _mlp_rows(blk['pre_attr'], u)
            pre_e = blk['pre_edge']
        else:
            vp, up, pre_e = v, u, None
        vi, vj = _edge_gather(vp, src, dst)
        (W1, b1) = blk['conv_edge'][0]
        c1 = up @ W1[96:128] + b1
        ec, e, me_sum = _conv_edge(e0, vi, vj, c1, pre_e, blk['conv_edge'])
        esA, esB = _scatter_add_32(ec, dst)
        (NW1, nb1) = blk['conv_node'][0]
        cn1 = up @ NW1[64:96] + nb1
        v, mv_sum = _conv_node(v0, vp, esA, esB, cntA, cntB, cn1,
                               blk['conv_node'])
        mean_v = mv_sum / N_NODES
        mean_e = me_sum / N_EDGES
        uc = _mlp_rows(blk['conv_attr'],
                       jnp.concatenate([up, mean_v, mean_e], axis=-1))
        u = uc + u0

    node_vec = _set2set(params['node_s2s'], v, T=2000)
    edge_vec = _set2set(params['edge_s2s'], e, T=4000)
    vec = jnp.concatenate([node_vec, edge_vec, u], axis=-1)
    out = _mlp_rows(params['output_proj'], vec, activate_last=False)
    return jnp.squeeze(out)


# speedup vs baseline: 2.1748x; 1.0880x over previous
import functools
import jax
import jax.numpy as jnp
from jax import lax
from jax.experimental import pallas as pl
from jax.experimental.pallas import tpu as pltpu
from jax.experimental.pallas import tpu_sc as plsc

N_NODES = 50000
N_EDGES = 800000
_LOG2 = 0.6931471805599453

_INTERP = False


def _sp2(x):
    return jax.nn.softplus(x) - _LOG2


def _mlp_rows(ps, x, activate_last=True):
    n = len(ps)
    for i, (W, b) in enumerate(ps):
        x = x @ W + b
        if i < n - 1 or activate_last:
            x = _sp2(x)
    return x


def _mlp2(x, p, T):
    (W1, b1), (W2, b2) = p
    N, Din = x.shape
    H = W1.shape[1]
    Dout = W2.shape[1]

    def body(x_ref, w1_ref, b1_ref, w2_ref, b2_ref, o_ref):
        h = _sp2(jnp.dot(x_ref[...], w1_ref[...],
                         preferred_element_type=jnp.float32) + b1_ref[...])
        o_ref[...] = _sp2(jnp.dot(h, w2_ref[...],
                                  preferred_element_type=jnp.float32) + b2_ref[...])

    return pl.pallas_call(
        body,
        grid=(N // T,),
        in_specs=[
            pl.BlockSpec((T, Din), lambda i: (i, 0)),
            pl.BlockSpec((Din, H), lambda i: (0, 0)),
            pl.BlockSpec((1, H), lambda i: (0, 0)),
            pl.BlockSpec((H, Dout), lambda i: (0, 0)),
            pl.BlockSpec((1, Dout), lambda i: (0, 0)),
        ],
        out_specs=pl.BlockSpec((T, Dout), lambda i: (i, 0)),
        out_shape=jax.ShapeDtypeStruct((N, Dout), jnp.float32),
        interpret=_INTERP,
    )(x, W1, b1.reshape(1, -1), W2, b2.reshape(1, -1))


def _node_encode(node_feat, emb, p, T=2000):
    (W1, b1), (W2, b2) = p
    n_elem, demb = emb.shape
    H = W1.shape[1]
    Dout = W2.shape[1]
    nf = node_feat.astype(jnp.int32).reshape(N_NODES, 1)

    def body(f_ref, emb_ref, w1_ref, b1_ref, w2_ref, b2_ref, o_ref):
        f = f_ref[...]
        ids = jax.lax.broadcasted_iota(jnp.int32, (T, n_elem), 1)
        oh = (f == ids).astype(jnp.float32)
        x = jnp.dot(oh, emb_ref[...], preferred_element_type=jnp.float32)
        h = _sp2(jnp.dot(x, w1_ref[...],
                         preferred_element_type=jnp.float32) + b1_ref[...])
        o_ref[...] = _sp2(jnp.dot(h, w2_ref[...],
                                  preferred_element_type=jnp.float32) + b2_ref[...])

    return pl.pallas_call(
        body,
        grid=(N_NODES // T,),
        in_specs=[
            pl.BlockSpec((T, 1), lambda i: (i, 0)),
            pl.BlockSpec((n_elem, demb), lambda i: (0, 0)),
            pl.BlockSpec((demb, H), lambda i: (0, 0)),
            pl.BlockSpec((1, H), lambda i: (0, 0)),
            pl.BlockSpec((H, Dout), lambda i: (0, 0)),
            pl.BlockSpec((1, Dout), lambda i: (0, 0)),
        ],
        out_specs=pl.BlockSpec((T, Dout), lambda i: (i, 0)),
        out_shape=jax.ShapeDtypeStruct((N_NODES, Dout), jnp.float32),
        interpret=_INTERP,
    )(nf, emb, W1, b1.reshape(1, -1), W2, b2.reshape(1, -1))


def _conv_edge(e0, vi, vj, c1, pre, cw, T=4000):
    (W1, _b1), (W2, b2), (W3, b3) = cw
    W1cut = W1[0:96]
    N = e0.shape[0]

    def body(*refs):
        if pre is not None:
            (e0_ref, vi_ref, vj_ref, c1_ref, w1_ref, w2_ref, b2_ref, w3_ref,
             b3_ref, p1w, p1b, p2w, p2b, ec_ref, er_ref, ms_ref) = refs
        else:
            (e0_ref, vi_ref, vj_ref, c1_ref, w1_ref, w2_ref, b2_ref, w3_ref,
             b3_ref, ec_ref, er_ref, ms_ref) = refs
        e0b = e0_ref[...]
        ep = e0b
        if pre is not None:
            ep = _sp2(jnp.dot(ep, p1w[...],
                              preferred_element_type=jnp.float32) + p1b[...])
            ep = _sp2(jnp.dot(ep, p2w[...],
                              preferred_element_type=jnp.float32) + p2b[...])
        x = jnp.concatenate([vi_ref[...], vj_ref[...], ep], axis=1)
        h = _sp2(jnp.dot(x, w1_ref[...],
                         preferred_element_type=jnp.float32) + c1_ref[...])
        h = _sp2(jnp.dot(h, w2_ref[...],
                         preferred_element_type=jnp.float32) + b2_ref[...])
        ec = _sp2(jnp.dot(h, w3_ref[...],
                          preferred_element_type=jnp.float32) + b3_ref[...])
        ec_ref[...] = ec
        er_ref[...] = ec + e0b
        i = pl.program_id(0)

        @pl.when(i == 0)
        def _():
            ms_ref[...] = jnp.zeros_like(ms_ref)

        ms_ref[...] += jnp.broadcast_to(
            jnp.sum(ec, axis=0, keepdims=True), ms_ref.shape)

    args = [e0, vi, vj, c1, W1cut, W2, b2.reshape(1, -1), W3, b3.reshape(1, -1)]
    in_specs = [
        pl.BlockSpec((T, 32), lambda i: (i, 0)),
        pl.BlockSpec((T, 32), lambda i: (i, 0)),
        pl.BlockSpec((T, 32), lambda i: (i, 0)),
        pl.BlockSpec((1, 64), lambda i: (0, 0)),
        pl.BlockSpec((96, 64), lambda i: (0, 0)),
        pl.BlockSpec((64, 64), lambda i: (0, 0)),
        pl.BlockSpec((1, 64), lambda i: (0, 0)),
        pl.BlockSpec((64, 32), lambda i: (0, 0)),
        pl.BlockSpec((1, 32), lambda i: (0, 0)),
    ]
    if pre is not None:
        (P1w, P1b), (P2w, P2b) = pre
        args += [P1w, P1b.reshape(1, -1), P2w, P2b.reshape(1, -1)]
        in_specs += [
            pl.BlockSpec((32, 64), lambda i: (0, 0)),
            pl.BlockSpec((1, 64), lambda i: (0, 0)),
            pl.BlockSpec((64, 32), lambda i: (0, 0)),
            pl.BlockSpec((1, 32), lambda i: (0, 0)),
        ]
    ec, er, ms = pl.pallas_call(
        body,
        grid=(N // T,),
        in_specs=in_specs,
        out_specs=[
            pl.BlockSpec((T, 32), lambda i: (i, 0)),
            pl.BlockSpec((T, 32), lambda i: (i, 0)),
            pl.BlockSpec((8, 32), lambda i: (0, 0)),
        ],
        out_shape=[
            jax.ShapeDtypeStruct((N, 32), jnp.float32),
            jax.ShapeDtypeStruct((N, 32), jnp.float32),
            jax.ShapeDtypeStruct((8, 32), jnp.float32),
        ],
        interpret=_INTERP,
    )(*args)
    return ec, er, ms[0:1]


def _conv_node(v0, vp, esA, esB, cntA, cntB, cn1, cw, T=2000):
    (W1, _b1), (W2, b2), (W3, b3) = cw
    W1cut = W1[0:64]

    def body(v0_ref, vp_ref, ea_ref, eb_ref, ca_ref, cb_ref, c1_ref,
             w1_ref, w2_ref, b2_ref, w3_ref, b3_ref, vr_ref, ms_ref):
        cnt = ca_ref[:, 0:1] + cb_ref[:, 0:1]
        ve = (ea_ref[...] + eb_ref[...]) / jnp.maximum(cnt, 1.0)
        x = jnp.concatenate([vp_ref[...], ve], axis=1)
        h = _sp2(jnp.dot(x, w1_ref[...],
                         preferred_element_type=jnp.float32) + c1_ref[...])
        h = _sp2(jnp.dot(h, w2_ref[...],
                         preferred_element_type=jnp.float32) + b2_ref[...])
        vc = _sp2(jnp.dot(h, w3_ref[...],
                          preferred_element_type=jnp.float32) + b3_ref[...])
        vr_ref[...] = vc + v0_ref[...]
        i = pl.program_id(0)

        @pl.when(i == 0)
        def _():
            ms_ref[...] = jnp.zeros_like(ms_ref)

        ms_ref[...] += jnp.broadcast_to(
            jnp.sum(vc, axis=0, keepdims=True), ms_ref.shape)

    vr, ms = pl.pallas_call(
        body,
        grid=(N_NODES // T,),
        in_specs=[
            pl.BlockSpec((T, 32), lambda i: (i, 0)),
            pl.BlockSpec((T, 32), lambda i: (i, 0)),
            pl.BlockSpec((T, 32), lambda i: (i, 0)),
            pl.BlockSpec((T, 32), lambda i: (i, 0)),
            pl.BlockSpec((T, 16), lambda i: (i, 0)),
            pl.BlockSpec((T, 16), lambda i: (i, 0)),
            pl.BlockSpec((1, 64), lambda i: (0, 0)),
            pl.BlockSpec((64, 64), lambda i: (0, 0)),
            pl.BlockSpec((64, 64), lambda i: (0, 0)),
            pl.BlockSpec((1, 64), lambda i: (0, 0)),
            pl.BlockSpec((64, 32), lambda i: (0, 0)),
            pl.BlockSpec((1, 32), lambda i: (0, 0)),
        ],
        out_specs=[
            pl.BlockSpec((T, 32), lambda i: (i, 0)),
            pl.BlockSpec((8, 32), lambda i: (0, 0)),
        ],
        out_shape=[
            jax.ShapeDtypeStruct((N_NODES, 32), jnp.float32),
            jax.ShapeDtypeStruct((8, 32), jnp.float32),
        ],
        interpret=_INTERP,
    )(v0, vp, esA, esB, cntA, cntB, cn1,
      W1cut, W2, b2.reshape(1, -1), W3, b3.reshape(1, -1))
    return vr, ms[0:1]


def _s2s_max(feat, q, T):
    N = feat.shape[0]

    def body(f_ref, q_ref, m_ref):
        en = jnp.sum(f_ref[...] * q_ref[...], axis=1, keepdims=True)
        mt = jnp.max(en, axis=0, keepdims=True)
        i = pl.program_id(0)

        @pl.when(i == 0)
        def _():
            m_ref[...] = jnp.full_like(m_ref, -jnp.inf)

        m_ref[...] = jnp.maximum(m_ref[...],
                                 jnp.broadcast_to(mt, m_ref.shape))

    return pl.pallas_call(
        body,
        grid=(N // T,),
        in_specs=[
            pl.BlockSpec((T, 32), lambda i: (i, 0)),
            pl.BlockSpec((1, 32), lambda i: (0, 0)),
        ],
        out_specs=pl.BlockSpec((8, 128), lambda i: (0, 0)),
        out_shape=jax.ShapeDtypeStruct((8, 128), jnp.float32),
        interpret=_INTERP,
    )(feat, q)


def _s2s_weighted(feat, q, m, T):
    N = feat.shape[0]

    def body(f_ref, q_ref, m_ref, r_ref, s_ref):
        f = f_ref[...]
        en = jnp.sum(f * q_ref[...], axis=1, keepdims=True)
        w = jnp.exp(en - m_ref[0:1, 0:1])
        r = jnp.sum(w * f, axis=0, keepdims=True)
        s = jnp.sum(w, axis=0, keepdims=True)
        i = pl.program_id(0)

        @pl.when(i == 0)
        def _():
            r_ref[...] = jnp.zeros_like(r_ref)
            s_ref[...] = jnp.zeros_like(s_ref)

        r_ref[...] += jnp.broadcast_to(r, r_ref.shape)
        s_ref[...] += jnp.broadcast_to(s, s_ref.shape)

    return pl.pallas_call(
        body,
        grid=(N // T,),
        in_specs=[
            pl.BlockSpec((T, 32), lambda i: (i, 0)),
            pl.BlockSpec((1, 32), lambda i: (0, 0)),
            pl.BlockSpec((8, 128), lambda i: (0, 0)),
        ],
        out_specs=[
            pl.BlockSpec((8, 32), lambda i: (0, 0)),
            pl.BlockSpec((8, 128), lambda i: (0, 0)),
        ],
        out_shape=[
            jax.ShapeDtypeStruct((8, 32), jnp.float32),
            jax.ShapeDtypeStruct((8, 128), jnp.float32),
        ],
        interpret=_INTERP,
    )(feat, q, m)


def _lstm(p, x, h, c):
    W_ih, W_hh, b_ih, b_hh = p
    gates = x @ W_ih.T + b_ih + h @ W_hh.T + b_hh
    i, f, g, o = jnp.split(gates, 4, axis=-1)
    c = jax.nn.sigmoid(f) * c + jax.nn.sigmoid(i) * jnp.tanh(g)
    h = jax.nn.sigmoid(o) * jnp.tanh(c)
    return h, c


def _set2set(p, feat, T):
    D = feat.shape[1]
    q_star = jnp.zeros((1, 2 * D), jnp.float32)
    h = jnp.zeros((1, D), jnp.float32)
    c = jnp.zeros((1, D), jnp.float32)
    for _ in range(2):
        h, c = _lstm(p, q_star, h, c)
        m = _s2s_max(feat, h, T)
        r, s = _s2s_weighted(feat, h, m, T)
        readout = r[0:1] / s[0, 0]
        q_star = jnp.concatenate([h, readout], axis=-1)
    return q_star


_NW = 32
_C = 128
_NCH = N_EDGES // _C
_JMAX = -(-_NCH // _NW)
_NB = N_NODES // 1000


def _wid():
    return lax.axis_index("s") * 2 + lax.axis_index("c")


def _edge_gather(vp, src, dst):
    mesh = plsc.VectorSubcoreMesh(core_axis_name="c", subcore_axis_name="s")

    K = 4

    @functools.partial(
        pl.kernel, mesh=mesh,
        out_type=[jax.ShapeDtypeStruct((N_EDGES, 32), jnp.float32),
                  jax.ShapeDtypeStruct((N_EDGES, 32), jnp.float32)],
        scratch_types=(
            [pltpu.VMEM((_C,), jnp.int32)] * (2 * K)
            + [pltpu.VMEM((_C, 32), jnp.float32)] * (2 * K)
            + [pltpu.SemaphoreType.DMA] * (2 * K + 1)
        ),
        compiler_params=pltpu.CompilerParams(use_tc_tiling_on_sc=False),
    )
    def k(vp_hbm, src_hbm, dst_hbm, vi_hbm, vj_hbm, *bufs):
        si = bufs[0:K]
        di = bufs[K:2 * K]
        ri = bufs[2 * K:3 * K]
        rj = bufs[3 * K:4 * K]
        sem_i = bufs[4 * K:5 * K]
        sem_g = bufs[5 * K:6 * K]
        sem_w = bufs[6 * K]
        w = _wid()

        def group(jbase, guard):
            offs = []
            for s in range(K):
                c = w + _NW * (jbase + s)
                off = c * _C
                offs.append((c, off))
                if guard:
                    @pl.when(c < _NCH)
                    def _(off=off, s=s):
                        pltpu.async_copy(src_hbm.at[pl.ds(off, _C)],
                                         si[s], sem_i[s])
                        pltpu.async_copy(dst_hbm.at[pl.ds(off, _C)],
                                         di[s], sem_i[s])
                else:
                    pltpu.async_copy(src_hbm.at[pl.ds(off, _C)], si[s],
                                     sem_i[s])
                    pltpu.async_copy(dst_hbm.at[pl.ds(off, _C)], di[s],
                                     sem_i[s])
            for s in range(K):
                c, off = offs[s]
                if guard:
                    @pl.when(c < _NCH)
                    def _(off=off, s=s):
                        pltpu.make_async_copy(src_hbm.at[pl.ds(off, _C)],
                                              si[s], sem_i[s]).wait()
                        pltpu.make_async_copy(dst_hbm.at[pl.ds(off, _C)],
                                              di[s], sem_i[s]).wait()
                        pltpu.async_copy(vp_hbm.at[si[s]], ri[s], sem_g[s])
                        pltpu.async_copy(vp_hbm.at[di[s]], rj[s], sem_g[s])
                else:
                    pltpu.make_async_copy(src_hbm.at[pl.ds(off, _C)],
                                          si[s], sem_i[s]).wait()
                    pltpu.make_async_copy(dst_hbm.at[pl.ds(off, _C)],
                                          di[s], sem_i[s]).wait()
                    pltpu.async_copy(vp_hbm.at[si[s]], ri[s], sem_g[s])
                    pltpu.async_copy(vp_hbm.at[di[s]], rj[s], sem_g[s])
            for s in range(K):
                c, off = offs[s]
                if guard:
                    @pl.when(c < _NCH)
                    def _(off=off, s=s):
                        pltpu.make_async_copy(vp_hbm.at[si[s]], ri[s],
                                              sem_g[s]).wait()
                        pltpu.make_async_copy(vp_hbm.at[di[s]], rj[s],
                                              sem_g[s]).wait()
                        pltpu.async_copy(ri[s], vi_hbm.at[pl.ds(off, _C)],
                                         sem_w)
                        pltpu.async_copy(rj[s], vj_hbm.at[pl.ds(off, _C)],
                                         sem_w)
                        pltpu.make_async_copy(ri[s],
                                              vi_hbm.at[pl.ds(off, _C)],
                                              sem_w).wait()
                        pltpu.make_async_copy(rj[s],
                                              vj_hbm.at[pl.ds(off, _C)],
                                              sem_w).wait()
                else:
                    pltpu.make_async_copy(vp_hbm.at[si[s]], ri[s],
                                          sem_g[s]).wait()
                    pltpu.make_async_copy(vp_hbm.at[di[s]], rj[s],
                                          sem_g[s]).wait()
                    pltpu.async_copy(ri[s], vi_hbm.at[pl.ds(off, _C)], sem_w)
                    pltpu.async_copy(rj[s], vj_hbm.at[pl.ds(off, _C)], sem_w)
            if not guard:
                for s in range(K):
                    c, off = offs[s]
                    pltpu.make_async_copy(ri[s], vi_hbm.at[pl.ds(off, _C)],
                                          sem_w).wait()
                    pltpu.make_async_copy(rj[s], vj_hbm.at[pl.ds(off, _C)],
                                          sem_w).wait()

        def body(jg, _):
            group(jg * K, guard=False)
            return 0

        n_full_groups = 192 // K
        lax.fori_loop(0, n_full_groups, body, 0)
        group(192, guard=True)

    return k(vp, src, dst)


def _sc_scatter_body(rows_hbm_or_none, idx_hbm, out_hbm, z_hbm, ones_hbm,
                     idx_b, rows_b, sem_i, shared):
    K = len(idx_b)
    w = _wid()
    sid = lax.axis_index("s")
    cid = lax.axis_index("c")

    if rows_hbm_or_none is None:
        for s in range(K):
            pltpu.sync_copy(ones_hbm, rows_b[s])

    for ci in range(4):
        b = sid + 16 * ci

        @pl.when(b < _NB)
        def _():
            pltpu.sync_copy(z_hbm, shared.at[pl.ds(b * 1000, 1000)])
    plsc.subcore_barrier()

    def group(jbase, guard):
        offs = []
        for s in range(K):
            c = w + _NW * (jbase + s)
            off = c * _C
            offs.append((c, off))

            def issue(off=off, s=s):
                pltpu.async_copy(idx_hbm.at[pl.ds(off, _C)], idx_b[s],
                                 sem_i[s])
                if rows_hbm_or_none is not None:
                    pltpu.async_copy(rows_hbm_or_none.at[pl.ds(off, _C)],
                                     rows_b[s], sem_i[s])
            if guard:
                pl.when(c < _NCH)(issue)
            else:
                issue()
        for s in range(K):
            c, off = offs[s]

            def drain(off=off, s=s):
                pltpu.make_async_copy(idx_hbm.at[pl.ds(off, _C)], idx_b[s],
                                      sem_i[s]).wait()
                if rows_hbm_or_none is not None:
                    pltpu.make_async_copy(
                        rows_hbm_or_none.at[pl.ds(off, _C)], rows_b[s],
                        sem_i[s]).wait()
                pltpu.sync_copy(rows_b[s], shared.at[idx_b[s]], add=True)
            if guard:
                pl.when(c < _NCH)(drain)
            else:
                drain()

    def body(jg, _):
        group(jg * K, guard=False)
        return 0

    lax.fori_loop(0, 192 // K, body, 0)
    group(192, guard=True)
    plsc.subcore_barrier()

    for ci in range(4):
        b = sid + 16 * ci

        @pl.when(b < _NB)
        def _():
            pltpu.sync_copy(shared.at[pl.ds(b * 1000, 1000)],
                            out_hbm.at[cid, pl.ds(b * 1000, 1000)])


def _scatter_add_32(rows, idx):
    mesh = plsc.VectorSubcoreMesh(core_axis_name="c", subcore_axis_name="s")
    z = jnp.zeros((1000, 32), jnp.float32)
    ones = jnp.ones((_C, 32), jnp.float32)

    K = 4

    @functools.partial(
        pl.kernel, mesh=mesh,
        out_type=jax.ShapeDtypeStruct((2, N_NODES, 32), jnp.float32),
        scratch_types=(
            [pltpu.VMEM((_C,), jnp.int32)] * K
            + [pltpu.VMEM((_C, 32), jnp.float32)] * K
            + [pltpu.SemaphoreType.DMA] * K
            + [pltpu.VMEM_SHARED((N_NODES, 32), jnp.float32)]
        ),
        compiler_params=pltpu.CompilerParams(use_tc_tiling_on_sc=False),
    )
    def k(rows_hbm, idx_hbm, z_hbm, ones_hbm, out_hbm, *bufs):
        _sc_scatter_body(rows_hbm, idx_hbm, out_hbm, z_hbm, ones_hbm,
                         bufs[0:K], bufs[K:2 * K], bufs[2 * K:3 * K],
                         bufs[3 * K])

    out = k(rows, idx, z, ones)
    return out[0], out[1]


def _seg_counts(idx):
    mesh = plsc.VectorSubcoreMesh(core_axis_name="c", subcore_axis_name="s")
    z = jnp.zeros((1000, 16), jnp.float32)
    ones = jnp.ones((_C, 16), jnp.float32)

    K = 4

    @functools.partial(
        pl.kernel, mesh=mesh,
        out_type=jax.ShapeDtypeStruct((2, N_NODES, 16), jnp.float32),
        scratch_types=(
            [pltpu.VMEM((_C,), jnp.int32)] * K
            + [pltpu.VMEM((_C, 16), jnp.float32)] * K
            + [pltpu.SemaphoreType.DMA] * K
            + [pltpu.VMEM_SHARED((N_NODES, 16), jnp.float32)]
        ),
        compiler_params=pltpu.CompilerParams(use_tc_tiling_on_sc=False),
    )
    def k(idx_hbm, z_hbm, ones_hbm, out_hbm, *bufs):
        _sc_scatter_body(None, idx_hbm, out_hbm, z_hbm, ones_hbm,
                         bufs[0:K], bufs[K:2 * K], bufs[2 * K:3 * K],
                         bufs[3 * K])

    out = k(idx, z, ones)
    return out[0], out[1]


def kernel(edge_feat, node_feat, edge_index, graph_attr, params):
    src = edge_index[0].astype(jnp.int32)
    dst = edge_index[1].astype(jnp.int32)

    e = _mlp2(edge_feat, params['edge_encoder'], T=4000)
    v = _node_encode(node_feat, params['node_embedding'],
                     params['node_encoder'])
    u = _mlp_rows(params['attr_encoder'], graph_attr)

    cntA, cntB = _seg_counts(dst)

    for blk in params['blocks']:
        e0, v0, u0 = e, v, u
        if blk['pre_edge'] is not None:
            vp = _mlp2(v, blk['pre_node'], T=2000)
            up = _mlp_rows(blk['pre_attr'], u)
            pre_e = blk['pre_edge']
        else:
            vp, up, pre_e = v, u, None
        vi, vj = _edge_gather(vp, src, dst)
        (W1, b1) = blk['conv_edge'][0]
        c1 = up @ W1[96:128] + b1
        ec, e, me_sum = _conv_edge(e0, vi, vj, c1, pre_e, blk['conv_edge'])
        esA, esB = _scatter_add_32(ec, dst)
        (NW1, nb1) = blk['conv_node'][0]
        cn1 = up @ NW1[64:96] + nb1
        v, mv_sum = _conv_node(v0, vp, esA, esB, cntA, cntB, cn1,
                               blk['conv_node'])
        mean_v = mv_sum / N_NODES
        mean_e = me_sum / N_EDGES
        uc = _mlp_rows(blk['conv_attr'],
                       jnp.concatenate([up, mean_v, mean_e], axis=-1))
        u = uc + u0

    node_vec = _set2set(params['node_s2s'], v, T=2000)
    edge_vec = _set2set(params['edge_s2s'], e, T=4000)
    vec = jnp.concatenate([node_vec, edge_vec, u], axis=-1)
    out = _mlp_rows(params['output_proj'], vec, activate_last=False)
    return jnp.squeeze(out)


# speedup vs baseline: 2.1824x; 1.0035x over previous
import functools
import jax
import jax.numpy as jnp
from jax import lax
from jax.experimental import pallas as pl
from jax.experimental.pallas import tpu as pltpu
from jax.experimental.pallas import tpu_sc as plsc

N_NODES = 50000
N_EDGES = 800000
_LOG2 = 0.6931471805599453

_INTERP = False


def _sp2(x):
    return jax.nn.softplus(x) - _LOG2


def _mlp_rows(ps, x, activate_last=True):
    n = len(ps)
    for i, (W, b) in enumerate(ps):
        x = x @ W + b
        if i < n - 1 or activate_last:
            x = _sp2(x)
    return x


def _mlp2(x, p, T):
    (W1, b1), (W2, b2) = p
    N, Din = x.shape
    H = W1.shape[1]
    Dout = W2.shape[1]

    def body(x_ref, w1_ref, b1_ref, w2_ref, b2_ref, o_ref):
        h = _sp2(jnp.dot(x_ref[...], w1_ref[...],
                         preferred_element_type=jnp.float32) + b1_ref[...])
        o_ref[...] = _sp2(jnp.dot(h, w2_ref[...],
                                  preferred_element_type=jnp.float32) + b2_ref[...])

    return pl.pallas_call(
        body,
        grid=(N // T,),
        in_specs=[
            pl.BlockSpec((T, Din), lambda i: (i, 0)),
            pl.BlockSpec((Din, H), lambda i: (0, 0)),
            pl.BlockSpec((1, H), lambda i: (0, 0)),
            pl.BlockSpec((H, Dout), lambda i: (0, 0)),
            pl.BlockSpec((1, Dout), lambda i: (0, 0)),
        ],
        out_specs=pl.BlockSpec((T, Dout), lambda i: (i, 0)),
        out_shape=jax.ShapeDtypeStruct((N, Dout), jnp.float32),
        interpret=_INTERP,
    )(x, W1, b1.reshape(1, -1), W2, b2.reshape(1, -1))


def _node_encode(node_feat, emb, p, T=2000):
    (W1, b1), (W2, b2) = p
    n_elem, demb = emb.shape
    H = W1.shape[1]
    Dout = W2.shape[1]
    nf = node_feat.astype(jnp.int32).reshape(N_NODES, 1)

    def body(f_ref, emb_ref, w1_ref, b1_ref, w2_ref, b2_ref, o_ref):
        f = f_ref[...]
        ids = jax.lax.broadcasted_iota(jnp.int32, (T, n_elem), 1)
        oh = (f == ids).astype(jnp.float32)
        x = jnp.dot(oh, emb_ref[...], preferred_element_type=jnp.float32)
        h = _sp2(jnp.dot(x, w1_ref[...],
                         preferred_element_type=jnp.float32) + b1_ref[...])
        o_ref[...] = _sp2(jnp.dot(h, w2_ref[...],
                                  preferred_element_type=jnp.float32) + b2_ref[...])

    return pl.pallas_call(
        body,
        grid=(N_NODES // T,),
        in_specs=[
            pl.BlockSpec((T, 1), lambda i: (i, 0)),
            pl.BlockSpec((n_elem, demb), lambda i: (0, 0)),
            pl.BlockSpec((demb, H), lambda i: (0, 0)),
            pl.BlockSpec((1, H), lambda i: (0, 0)),
            pl.BlockSpec((H, Dout), lambda i: (0, 0)),
            pl.BlockSpec((1, Dout), lambda i: (0, 0)),
        ],
        out_specs=pl.BlockSpec((T, Dout), lambda i: (i, 0)),
        out_shape=jax.ShapeDtypeStruct((N_NODES, Dout), jnp.float32),
        interpret=_INTERP,
    )(nf, emb, W1, b1.reshape(1, -1), W2, b2.reshape(1, -1))


def _conv_edge(e0, vi, vj, c1, pre, cw, T=4000):
    (W1, _b1), (W2, b2), (W3, b3) = cw
    W1cut = W1[0:96]
    N = e0.shape[0]

    def body(*refs):
        if pre is not None:
            (e0_ref, vi_ref, vj_ref, c1_ref, w1_ref, w2_ref, b2_ref, w3_ref,
             b3_ref, p1w, p1b, p2w, p2b, ec_ref, er_ref, ms_ref) = refs
        else:
            (e0_ref, vi_ref, vj_ref, c1_ref, w1_ref, w2_ref, b2_ref, w3_ref,
             b3_ref, ec_ref, er_ref, ms_ref) = refs
        e0b = e0_ref[...]
        ep = e0b
        if pre is not None:
            ep = _sp2(jnp.dot(ep, p1w[...],
                              preferred_element_type=jnp.float32) + p1b[...])
            ep = _sp2(jnp.dot(ep, p2w[...],
                              preferred_element_type=jnp.float32) + p2b[...])
        x = jnp.concatenate([vi_ref[...], vj_ref[...], ep], axis=1)
        h = _sp2(jnp.dot(x, w1_ref[...],
                         preferred_element_type=jnp.float32) + c1_ref[...])
        h = _sp2(jnp.dot(h, w2_ref[...],
                         preferred_element_type=jnp.float32) + b2_ref[...])
        ec = _sp2(jnp.dot(h, w3_ref[...],
                          preferred_element_type=jnp.float32) + b3_ref[...])
        ec_ref[...] = ec
        er_ref[...] = ec + e0b
        i = pl.program_id(0)

        @pl.when(i == 0)
        def _():
            ms_ref[...] = jnp.zeros_like(ms_ref)

        ms_ref[...] += jnp.broadcast_to(
            jnp.sum(ec, axis=0, keepdims=True), ms_ref.shape)

    args = [e0, vi, vj, c1, W1cut, W2, b2.reshape(1, -1), W3, b3.reshape(1, -1)]
    in_specs = [
        pl.BlockSpec((T, 32), lambda i: (i, 0)),
        pl.BlockSpec((T, 32), lambda i: (i, 0)),
        pl.BlockSpec((T, 32), lambda i: (i, 0)),
        pl.BlockSpec((1, 64), lambda i: (0, 0)),
        pl.BlockSpec((96, 64), lambda i: (0, 0)),
        pl.BlockSpec((64, 64), lambda i: (0, 0)),
        pl.BlockSpec((1, 64), lambda i: (0, 0)),
        pl.BlockSpec((64, 32), lambda i: (0, 0)),
        pl.BlockSpec((1, 32), lambda i: (0, 0)),
    ]
    if pre is not None:
        (P1w, P1b), (P2w, P2b) = pre
        args += [P1w, P1b.reshape(1, -1), P2w, P2b.reshape(1, -1)]
        in_specs += [
            pl.BlockSpec((32, 64), lambda i: (0, 0)),
            pl.BlockSpec((1, 64), lambda i: (0, 0)),
            pl.BlockSpec((64, 32), lambda i: (0, 0)),
            pl.BlockSpec((1, 32), lambda i: (0, 0)),
        ]
    ec, er, ms = pl.pallas_call(
        body,
        grid=(N // T,),
        in_specs=in_specs,
        out_specs=[
            pl.BlockSpec((T, 32), lambda i: (i, 0)),
            pl.BlockSpec((T, 32), lambda i: (i, 0)),
            pl.BlockSpec((8, 32), lambda i: (0, 0)),
        ],
        out_shape=[
            jax.ShapeDtypeStruct((N, 32), jnp.float32),
            jax.ShapeDtypeStruct((N, 32), jnp.float32),
            jax.ShapeDtypeStruct((8, 32), jnp.float32),
        ],
        interpret=_INTERP,
    )(*args)
    return ec, er, ms[0:1]


def _conv_node(v0, vp, esA, esB, cntA, cntB, cn1, cw, pre_next, T=2000):
    (W1, _b1), (W2, b2), (W3, b3) = cw
    W1cut = W1[0:64]

    def body(*refs):
        if pre_next is not None:
            (v0_ref, vp_ref, ea_ref, eb_ref, ca_ref, cb_ref, c1_ref,
             w1_ref, w2_ref, b2_ref, w3_ref, b3_ref, q1w, q1b, q2w, q2b,
             vr_ref, ms_ref, vpn_ref) = refs
        else:
            (v0_ref, vp_ref, ea_ref, eb_ref, ca_ref, cb_ref, c1_ref,
             w1_ref, w2_ref, b2_ref, w3_ref, b3_ref,
             vr_ref, ms_ref) = refs
        cnt = ca_ref[:, 0:1] + cb_ref[:, 0:1]
        ve = (ea_ref[...] + eb_ref[...]) / jnp.maximum(cnt, 1.0)
        x = jnp.concatenate([vp_ref[...], ve], axis=1)
        h = _sp2(jnp.dot(x, w1_ref[...],
                         preferred_element_type=jnp.float32) + c1_ref[...])
        h = _sp2(jnp.dot(h, w2_ref[...],
                         preferred_element_type=jnp.float32) + b2_ref[...])
        vc = _sp2(jnp.dot(h, w3_ref[...],
                          preferred_element_type=jnp.float32) + b3_ref[...])
        vr = vc + v0_ref[...]
        vr_ref[...] = vr
        if pre_next is not None:
            p = _sp2(jnp.dot(vr, q1w[...],
                             preferred_element_type=jnp.float32) + q1b[...])
            vpn_ref[...] = _sp2(jnp.dot(p, q2w[...],
                                        preferred_element_type=jnp.float32)
                                + q2b[...])
        i = pl.program_id(0)

        @pl.when(i == 0)
        def _():
            ms_ref[...] = jnp.zeros_like(ms_ref)

        ms_ref[...] += jnp.broadcast_to(
            jnp.sum(vc, axis=0, keepdims=True), ms_ref.shape)

    args = [v0, vp, esA, esB, cntA, cntB, cn1,
            W1cut, W2, b2.reshape(1, -1), W3, b3.reshape(1, -1)]
    in_specs = [
        pl.BlockSpec((T, 32), lambda i: (i, 0)),
        pl.BlockSpec((T, 32), lambda i: (i, 0)),
        pl.BlockSpec((T, 32), lambda i: (i, 0)),
        pl.BlockSpec((T, 32), lambda i: (i, 0)),
        pl.BlockSpec((T, 16), lambda i: (i, 0)),
        pl.BlockSpec((T, 16), lambda i: (i, 0)),
        pl.BlockSpec((1, 64), lambda i: (0, 0)),
        pl.BlockSpec((64, 64), lambda i: (0, 0)),
        pl.BlockSpec((64, 64), lambda i: (0, 0)),
        pl.BlockSpec((1, 64), lambda i: (0, 0)),
        pl.BlockSpec((64, 32), lambda i: (0, 0)),
        pl.BlockSpec((1, 32), lambda i: (0, 0)),
    ]
    out_specs = [
        pl.BlockSpec((T, 32), lambda i: (i, 0)),
        pl.BlockSpec((8, 32), lambda i: (0, 0)),
    ]
    out_shape = [
        jax.ShapeDtypeStruct((N_NODES, 32), jnp.float32),
        jax.ShapeDtypeStruct((8, 32), jnp.float32),
    ]
    if pre_next is not None:
        (Q1w, Q1b), (Q2w, Q2b) = pre_next
        args += [Q1w, Q1b.reshape(1, -1), Q2w, Q2b.reshape(1, -1)]
        in_specs += [
            pl.BlockSpec((32, 64), lambda i: (0, 0)),
            pl.BlockSpec((1, 64), lambda i: (0, 0)),
            pl.BlockSpec((64, 32), lambda i: (0, 0)),
            pl.BlockSpec((1, 32), lambda i: (0, 0)),
        ]
        out_specs.append(pl.BlockSpec((T, 32), lambda i: (i, 0)))
        out_shape.append(jax.ShapeDtypeStruct((N_NODES, 32), jnp.float32))
    res = pl.pallas_call(
        body,
        grid=(N_NODES // T,),
        in_specs=in_specs,
        out_specs=out_specs,
        out_shape=out_shape,
        interpret=_INTERP,
    )(*args)
    if pre_next is not None:
        vr, ms, vpn = res
        return vr, ms[0:1], vpn
    vr, ms = res
    return vr, ms[0:1], None


def _set2set(p, feat, T):
    W_ih, W_hh, b_ih, b_hh = p
    WihT = W_ih.T
    WhhT = W_hh.T
    bias = (b_ih + b_hh).reshape(1, -1)
    N = feat.shape[0]
    NT = N // T

    def body(f_ref, wi_ref, wh_ref, b_ref, o_ref,
             h_s, c_s, q_s, qs_s, m_s, r_s, s_s):
        it = pl.program_id(0)
        ph = pl.program_id(1)
        t = pl.program_id(2)

        @pl.when(jnp.logical_and(jnp.logical_and(ph == 0, t == 0), it == 0))
        def _():
            h_s[...] = jnp.zeros_like(h_s)
            c_s[...] = jnp.zeros_like(c_s)
            qs_s[...] = jnp.zeros_like(qs_s)

        @pl.when(jnp.logical_and(ph == 0, t == 0))
        def _():
            gates = (jnp.dot(qs_s[0:1], wi_ref[...],
                             preferred_element_type=jnp.float32)
                     + jnp.dot(h_s[0:1], wh_ref[...],
                               preferred_element_type=jnp.float32)
                     + b_ref[...])
            ig = jax.nn.sigmoid(gates[:, 0:32])
            fg = jax.nn.sigmoid(gates[:, 32:64])
            gg = jnp.tanh(gates[:, 64:96])
            og = jax.nn.sigmoid(gates[:, 96:128])
            cn = fg * c_s[0:1] + ig * gg
            hn = og * jnp.tanh(cn)
            c_s[...] = jnp.broadcast_to(cn, c_s.shape)
            h_s[...] = jnp.broadcast_to(hn, h_s.shape)
            q_s[...] = jnp.broadcast_to(hn, q_s.shape)
            m_s[...] = jnp.full_like(m_s, -jnp.inf)
            r_s[...] = jnp.zeros_like(r_s)
            s_s[...] = jnp.zeros_like(s_s)

        f = f_ref[...]
        en = jnp.sum(f * q_s[0:1], axis=1, keepdims=True)

        @pl.when(ph == 0)
        def _():
            mt = jnp.max(en, axis=0, keepdims=True)
            m_s[...] = jnp.maximum(m_s[...],
                                   jnp.broadcast_to(mt, m_s.shape))

        @pl.when(ph == 1)
        def _():
            w = jnp.exp(en - m_s[0:1, 0:1])
            r_s[...] += jnp.broadcast_to(
                jnp.sum(w * f, axis=0, keepdims=True), r_s.shape)
            s_s[...] += jnp.broadcast_to(
                jnp.sum(w, axis=0, keepdims=True), s_s.shape)

        @pl.when(jnp.logical_and(ph == 1, t == NT - 1))
        def _():
            ro = r_s[0:1] / s_s[0:1, 0:1]
            qs = jnp.concatenate([q_s[0:1], ro], axis=1)
            qs_s[...] = jnp.broadcast_to(qs, qs_s.shape)

            @pl.when(it == 1)
            def _():
                o_ref[...] = jnp.broadcast_to(qs, o_ref.shape)

    out = pl.pallas_call(
        body,
        grid=(2, 2, NT),
        in_specs=[
            pl.BlockSpec((T, 32), lambda it, ph, t: (t, 0)),
            pl.BlockSpec((64, 128), lambda it, ph, t: (0, 0)),
            pl.BlockSpec((32, 128), lambda it, ph, t: (0, 0)),
            pl.BlockSpec((1, 128), lambda it, ph, t: (0, 0)),
        ],
        out_specs=pl.BlockSpec((8, 64), lambda it, ph, t: (0, 0)),
        out_shape=jax.ShapeDtypeStruct((8, 64), jnp.float32),
        scratch_shapes=[
            pltpu.VMEM((8, 32), jnp.float32),
            pltpu.VMEM((8, 32), jnp.float32),
            pltpu.VMEM((8, 32), jnp.float32),
            pltpu.VMEM((8, 64), jnp.float32),
            pltpu.VMEM((8, 128), jnp.float32),
            pltpu.VMEM((8, 32), jnp.float32),
            pltpu.VMEM((8, 128), jnp.float32),
        ],
        interpret=_INTERP,
    )(feat, WihT, WhhT, bias)
    return out[0:1]


_NW = 32
_C = 128
_NCH = N_EDGES // _C
_JMAX = -(-_NCH // _NW)
_NB = N_NODES // 1000


def _wid():
    return lax.axis_index("s") * 2 + lax.axis_index("c")


def _edge_gather(vp, src, dst):
    mesh = plsc.VectorSubcoreMesh(core_axis_name="c", subcore_axis_name="s")

    K = 4

    @functools.partial(
        pl.kernel, mesh=mesh,
        out_type=[jax.ShapeDtypeStruct((N_EDGES, 32), jnp.float32),
                  jax.ShapeDtypeStruct((N_EDGES, 32), jnp.float32)],
        scratch_types=(
            [pltpu.VMEM((_C,), jnp.int32)] * (2 * K)
            + [pltpu.VMEM((_C, 32), jnp.float32)] * (2 * K)
            + [pltpu.SemaphoreType.DMA] * (2 * K + 1)
        ),
        compiler_params=pltpu.CompilerParams(use_tc_tiling_on_sc=False),
    )
    def k(vp_hbm, src_hbm, dst_hbm, vi_hbm, vj_hbm, *bufs):
        si = bufs[0:K]
        di = bufs[K:2 * K]
        ri = bufs[2 * K:3 * K]
        rj = bufs[3 * K:4 * K]
        sem_i = bufs[4 * K:5 * K]
        sem_g = bufs[5 * K:6 * K]
        sem_w = bufs[6 * K]
        w = _wid()

        def group(jbase, guard):
            offs = []
            for s in range(K):
                c = w + _NW * (jbase + s)
                off = c * _C
                offs.append((c, off))
                if guard:
                    @pl.when(c < _NCH)
                    def _(off=off, s=s):
                        pltpu.async_copy(src_hbm.at[pl.ds(off, _C)],
                                         si[s], sem_i[s])
                        pltpu.async_copy(dst_hbm.at[pl.ds(off, _C)],
                                         di[s], sem_i[s])
                else:
                    pltpu.async_copy(src_hbm.at[pl.ds(off, _C)], si[s],
                                     sem_i[s])
                    pltpu.async_copy(dst_hbm.at[pl.ds(off, _C)], di[s],
                                     sem_i[s])
            for s in range(K):
                c, off = offs[s]
                if guard:
                    @pl.when(c < _NCH)
                    def _(off=off, s=s):
                        pltpu.make_async_copy(src_hbm.at[pl.ds(off, _C)],
                                              si[s], sem_i[s]).wait()
                        pltpu.make_async_copy(dst_hbm.at[pl.ds(off, _C)],
                                              di[s], sem_i[s]).wait()
                        pltpu.async_copy(vp_hbm.at[si[s]], ri[s], sem_g[s])
                        pltpu.async_copy(vp_hbm.at[di[s]], rj[s], sem_g[s])
                else:
                    pltpu.make_async_copy(src_hbm.at[pl.ds(off, _C)],
                                          si[s], sem_i[s]).wait()
                    pltpu.make_async_copy(dst_hbm.at[pl.ds(off, _C)],
                                          di[s], sem_i[s]).wait()
                    pltpu.async_copy(vp_hbm.at[si[s]], ri[s], sem_g[s])
                    pltpu.async_copy(vp_hbm.at[di[s]], rj[s], sem_g[s])
            for s in range(K):
                c, off = offs[s]
                if guard:
                    @pl.when(c < _NCH)
                    def _(off=off, s=s):
                        pltpu.make_async_copy(vp_hbm.at[si[s]], ri[s],
                                              sem_g[s]).wait()
                        pltpu.make_async_copy(vp_hbm.at[di[s]], rj[s],
                                              sem_g[s]).wait()
                        pltpu.async_copy(ri[s], vi_hbm.at[pl.ds(off, _C)],
                                         sem_w)
                        pltpu.async_copy(rj[s], vj_hbm.at[pl.ds(off, _C)],
                                         sem_w)
                        pltpu.make_async_copy(ri[s],
                                              vi_hbm.at[pl.ds(off, _C)],
                                              sem_w).wait()
                        pltpu.make_async_copy(rj[s],
                                              vj_hbm.at[pl.ds(off, _C)],
                                              sem_w).wait()
                else:
                    pltpu.make_async_copy(vp_hbm.at[si[s]], ri[s],
                                          sem_g[s]).wait()
                    pltpu.make_async_copy(vp_hbm.at[di[s]], rj[s],
                                          sem_g[s]).wait()
                    pltpu.async_copy(ri[s], vi_hbm.at[pl.ds(off, _C)], sem_w)
                    pltpu.async_copy(rj[s], vj_hbm.at[pl.ds(off, _C)], sem_w)
            if not guard:
                for s in range(K):
                    c, off = offs[s]
                    pltpu.make_async_copy(ri[s], vi_hbm.at[pl.ds(off, _C)],
                                          sem_w).wait()
                    pltpu.make_async_copy(rj[s], vj_hbm.at[pl.ds(off, _C)],
                                          sem_w).wait()

        def body(jg, _):
            group(jg * K, guard=False)
            return 0

        n_full_groups = 192 // K
        lax.fori_loop(0, n_full_groups, body, 0)
        group(192, guard=True)

    return k(vp, src, dst)


def _sc_scatter_body(rows_hbm_or_none, idx_hbm, out_hbm, z_hbm, ones_hbm,
                     idx_b, rows_b, sem_i, shared):
    K = len(idx_b)
    w = _wid()
    sid = lax.axis_index("s")
    cid = lax.axis_index("c")

    if rows_hbm_or_none is None:
        for s in range(K):
            pltpu.sync_copy(ones_hbm, rows_b[s])

    for ci in range(4):
        b = sid + 16 * ci

        @pl.when(b < _NB)
        def _():
            pltpu.sync_copy(z_hbm, shared.at[pl.ds(b * 1000, 1000)])
    plsc.subcore_barrier()

    def group(jbase, guard):
        offs = []
        for s in range(K):
            c = w + _NW * (jbase + s)
            off = c * _C
            offs.append((c, off))

            def issue(off=off, s=s):
                pltpu.async_copy(idx_hbm.at[pl.ds(off, _C)], idx_b[s],
                                 sem_i[s])
                if rows_hbm_or_none is not None:
                    pltpu.async_copy(rows_hbm_or_none.at[pl.ds(off, _C)],
                                     rows_b[s], sem_i[s])
            if guard:
                pl.when(c < _NCH)(issue)
            else:
                issue()
        for s in range(K):
            c, off = offs[s]

            def drain(off=off, s=s):
                pltpu.make_async_copy(idx_hbm.at[pl.ds(off, _C)], idx_b[s],
                                      sem_i[s]).wait()
                if rows_hbm_or_none is not None:
                    pltpu.make_async_copy(
                        rows_hbm_or_none.at[pl.ds(off, _C)], rows_b[s],
                        sem_i[s]).wait()
                pltpu.sync_copy(rows_b[s], shared.at[idx_b[s]], add=True)
            if guard:
                pl.when(c < _NCH)(drain)
            else:
                drain()

    def body(jg, _):
        group(jg * K, guard=False)
        return 0

    lax.fori_loop(0, 192 // K, body, 0)
    group(192, guard=True)
    plsc.subcore_barrier()

    for ci in range(4):
        b = sid + 16 * ci

        @pl.when(b < _NB)
        def _():
            pltpu.sync_copy(shared.at[pl.ds(b * 1000, 1000)],
                            out_hbm.at[cid, pl.ds(b * 1000, 1000)])


def _scatter_add_32(rows, idx):
    mesh = plsc.VectorSubcoreMesh(core_axis_name="c", subcore_axis_name="s")
    z = jnp.zeros((1000, 32), jnp.float32)
    ones = jnp.ones((_C, 32), jnp.float32)

    K = 4

    @functools.partial(
        pl.kernel, mesh=mesh,
        out_type=jax.ShapeDtypeStruct((2, N_NODES, 32), jnp.float32),
        scratch_types=(
            [pltpu.VMEM((_C,), jnp.int32)] * K
            + [pltpu.VMEM((_C, 32), jnp.float32)] * K
            + [pltpu.SemaphoreType.DMA] * K
            + [pltpu.VMEM_SHARED((N_NODES, 32), jnp.float32)]
        ),
        compiler_params=pltpu.CompilerParams(use_tc_tiling_on_sc=False),
    )
    def k(rows_hbm, idx_hbm, z_hbm, ones_hbm, out_hbm, *bufs):
        _sc_scatter_body(rows_hbm, idx_hbm, out_hbm, z_hbm, ones_hbm,
                         bufs[0:K], bufs[K:2 * K], bufs[2 * K:3 * K],
                         bufs[3 * K])

    out = k(rows, idx, z, ones)
    return out[0], out[1]


def _seg_counts(idx):
    mesh = plsc.VectorSubcoreMesh(core_axis_name="c", subcore_axis_name="s")
    z = jnp.zeros((1000, 16), jnp.float32)
    ones = jnp.ones((_C, 16), jnp.float32)

    K = 4

    @functools.partial(
        pl.kernel, mesh=mesh,
        out_type=jax.ShapeDtypeStruct((2, N_NODES, 16), jnp.float32),
        scratch_types=(
            [pltpu.VMEM((_C,), jnp.int32)] * K
            + [pltpu.VMEM((_C, 16), jnp.float32)] * K
            + [pltpu.SemaphoreType.DMA] * K
            + [pltpu.VMEM_SHARED((N_NODES, 16), jnp.float32)]
        ),
        compiler_params=pltpu.CompilerParams(use_tc_tiling_on_sc=False),
    )
    def k(idx_hbm, z_hbm, ones_hbm, out_hbm, *bufs):
        _sc_scatter_body(None, idx_hbm, out_hbm, z_hbm, ones_hbm,
                         bufs[0:K], bufs[K:2 * K], bufs[2 * K:3 * K],
                         bufs[3 * K])

    out = k(idx, z, ones)
    return out[0], out[1]


def kernel(edge_feat, node_feat, edge_index, graph_attr, params):
    src = edge_index[0].astype(jnp.int32)
    dst = edge_index[1].astype(jnp.int32)

    e = _mlp2(edge_feat, params['edge_encoder'], T=4000)
    v = _node_encode(node_feat, params['node_embedding'],
                     params['node_encoder'])
    u = _mlp_rows(params['attr_encoder'], graph_attr)

    cntA, cntB = _seg_counts(dst)

    blocks = params['blocks']
    vp_next = None
    for bi, blk in enumerate(blocks):
        e0, v0, u0 = e, v, u
        if blk['pre_edge'] is not None:
            vp = vp_next
            up = _mlp_rows(blk['pre_attr'], u)
            pre_e = blk['pre_edge']
        else:
            vp, up, pre_e = v, u, None
        vi, vj = _edge_gather(vp, src, dst)
        (W1, b1) = blk['conv_edge'][0]
        c1 = up @ W1[96:128] + b1
        ec, e, me_sum = _conv_edge(e0, vi, vj, c1, pre_e, blk['conv_edge'])
        esA, esB = _scatter_add_32(ec, dst)
        (NW1, nb1) = blk['conv_node'][0]
        cn1 = up @ NW1[64:96] + nb1
        nxt = blocks[bi + 1]['pre_node'] if bi + 1 < len(blocks) else None
        v, mv_sum, vp_next = _conv_node(v0, vp, esA, esB, cntA, cntB, cn1,
                                        blk['conv_node'], nxt)
        mean_v = mv_sum / N_NODES
        mean_e = me_sum / N_EDGES
        uc = _mlp_rows(blk['conv_attr'],
                       jnp.concatenate([up, mean_v, mean_e], axis=-1))
        u = uc + u0

    node_vec = _set2set(params['node_s2s'], v, T=2000)
    edge_vec = _set2set(params['edge_s2s'], e, T=4000)
    vec = jnp.concatenate([node_vec, edge_vec, u], axis=-1)
    out = _mlp_rows(params['output_proj'], vec, activate_last=False)
    return jnp.squeeze(out)
